# baseline (speedup 1.0000x reference)
import functools

import jax
import jax.numpy as jnp
from jax import lax
from jax.experimental import pallas as pl
from jax.experimental.pallas import tpu as pltpu
from jax.experimental.pallas import tpu_sc as plsc

E = 16
H = 768
I = 512
I_SH = 256
T = 2048
AUX_COEFF = 0.01

BT = 256
BTSHIFT = 8
NB = 24
P = NB * BT
BT3 = 512

_NC = 2
_NS = 16
_NW = _NC * _NS
_TPW = T // _NW


RC = 4
RT = T // RC


def _router_body(x_ref, rw_ref, dest_ref, be_ref, nbv_ref, aux_ref, lg_ref):
    g = pl.program_id(0)
    xf = x_ref[...]
    rw = rw_ref[...]
    lg_ref[pl.ds(g * RT, RT), :] = lax.dot_general(
        xf, rw, (((1,), (1,)), ((), ())),
        preferred_element_type=jnp.float32)

    @pl.when(g == RC - 1)
    def _meta():
        logits = lg_ref[...]
        lmax = jnp.max(logits, axis=1, keepdims=True)
        ex = jnp.exp(logits - lmax)
        probs = ex / jnp.sum(ex, axis=1, keepdims=True)

        ei = lax.broadcasted_iota(jnp.int32, (T, E), 1)
        is_max = logits == lmax
        eid = jnp.min(jnp.where(is_max, ei, E), axis=1, keepdims=True)
        onehot = (ei == eid).astype(jnp.int32)

        counts = jnp.sum(onehot, axis=0, keepdims=True)

        cum = onehot
        k = 1
        while k < T:
            cum = cum + jnp.concatenate(
                [jnp.zeros((k, E), jnp.int32), cum[: T - k, :]], axis=0)
            k *= 2
        rank = jnp.sum(onehot * cum, axis=1, keepdims=True) - 1

        nb_e = lax.shift_right_logical(counts + (BT - 1), BTSHIFT)
        cnb = nb_e
        k = 1
        while k < E:
            cnb = cnb + jnp.concatenate(
                [jnp.zeros((1, k), jnp.int32), cnb[:, : E - k]], axis=1)
            k *= 2
        poffset = lax.shift_left(cnb - nb_e, BTSHIFT)
        nb_total = jnp.max(cnb, axis=1, keepdims=True)

        dest = jnp.sum(onehot * poffset, axis=1, keepdims=True) + rank

        e1 = lax.broadcasted_iota(jnp.int32, (1, E), 1)
        last_e = jnp.max(jnp.where(counts > 0, e1, 0), axis=1, keepdims=True)
        g_iota = lax.broadcasted_iota(jnp.int32, (NB, E), 0)
        cnb_b = jnp.broadcast_to(cnb, (NB, E))
        raw = jnp.sum((g_iota >= cnb_b).astype(jnp.int32), axis=1,
                      keepdims=True)
        be = jnp.minimum(raw, last_e)

        fsum = jnp.sum(counts.astype(jnp.float32) * jnp.sum(probs, axis=0,
                                                            keepdims=True),
                       axis=1, keepdims=True)
        aux = (AUX_COEFF * E / (T * T)) * fsum

        dest_ref[...] = dest
        be_ref[...] = be
        nbv_ref[...] = nb_total
        aux_ref[...] = aux


def _router_call(flat, router_w):
    return pl.pallas_call(
        _router_body,
        grid=(RC,),
        in_specs=[
            pl.BlockSpec((RT, H), lambda g: (g, 0)),
            pl.BlockSpec((E, H), lambda g: (0, 0)),
        ],
        out_specs=(
            pl.BlockSpec((T, 1), lambda g: (0, 0)),
            pl.BlockSpec((NB, 1), lambda g: (0, 0)),
            pl.BlockSpec((1, 1), lambda g: (0, 0)),
            pl.BlockSpec((1, 1), lambda g: (0, 0)),
        ),
        out_shape=(
            jax.ShapeDtypeStruct((T, 1), jnp.int32),
            jax.ShapeDtypeStruct((NB, 1), jnp.int32),
            jax.ShapeDtypeStruct((1, 1), jnp.int32),
            jax.ShapeDtypeStruct((1, 1), jnp.float32),
        ),
        scratch_shapes=[pltpu.VMEM((T, E), jnp.float32)],
    )(flat, router_w)


@functools.lru_cache(maxsize=None)
def _sc_scatter_kernel():
    mesh = plsc.VectorSubcoreMesh(core_axis_name="c", subcore_axis_name="s")

    @functools.partial(
        pl.kernel,
        mesh=mesh,
        out_type=jax.ShapeDtypeStruct((P, H), jnp.float32),
        scratch_types=[
            pltpu.VMEM((_TPW,), jnp.int32),
            pltpu.VMEM((_TPW, H), jnp.float32),
            pltpu.SemaphoreType.DMA,
        ],
    )
    def _sc_scatter(x_hbm, dest_hbm, out_hbm, idx_v, rows_v, sem):
        wid = lax.axis_index("s") * _NC + lax.axis_index("c")
        base = wid * _TPW
        pltpu.sync_copy(dest_hbm.at[pl.ds(base, _TPW)], idx_v)
        pltpu.sync_copy(x_hbm.at[pl.ds(base, _TPW)], rows_v)
        pltpu.async_copy(rows_v, out_hbm.at[idx_v], sem).wait()

    return _sc_scatter


@functools.lru_cache(maxsize=None)
def _sc_gather_kernel():
    mesh = plsc.VectorSubcoreMesh(core_axis_name="c", subcore_axis_name="s")

    @functools.partial(
        pl.kernel,
        mesh=mesh,
        out_type=jax.ShapeDtypeStruct((T, H), jnp.float32),
        scratch_types=[
            pltpu.VMEM((_TPW,), jnp.int32),
            pltpu.VMEM((_TPW, H), jnp.float32),
            pltpu.SemaphoreType.DMA,
        ],
    )
    def _sc_gather(ys_hbm, dest_hbm, out_hbm, idx_v, rows_v, sem):
        wid = lax.axis_index("s") * _NC + lax.axis_index("c")
        base = wid * _TPW
        pltpu.sync_copy(dest_hbm.at[pl.ds(base, _TPW)], idx_v)
        pltpu.async_copy(ys_hbm.at[idx_v], rows_v, sem).wait()
        pltpu.sync_copy(rows_v, out_hbm.at[pl.ds(base, _TPW)])

    return _sc_gather


def _ffn_body(be_ref, nbv_ref, xs_ref, gw_ref, uw_ref, dw_ref, ys_ref):
    gb = pl.program_id(0)

    @pl.when(gb < nbv_ref[0])
    def _():
        xb = xs_ref[...]
        gw = gw_ref[0]
        uw = uw_ref[0]
        dw = dw_ref[0]
        g = lax.dot_general(xb, gw, (((1,), (1,)), ((), ())),
                            preferred_element_type=jnp.float32)
        u = lax.dot_general(xb, uw, (((1,), (1,)), ((), ())),
                            preferred_element_type=jnp.float32)
        h = g * lax.logistic(g) * u
        y = lax.dot_general(h, dw, (((1,), (1,)), ((), ())),
                            preferred_element_type=jnp.float32)
        ys_ref[...] = y


def _ffn_call(be, nbv, xs, gate_w, up_w, down_w):
    grid_spec = pltpu.PrefetchScalarGridSpec(
        num_scalar_prefetch=2,
        grid=(NB,),
        in_specs=[
            pl.BlockSpec((BT, H),
                         lambda g, be_r, nb_r: (jnp.minimum(g, nb_r[0] - 1), 0)),
            pl.BlockSpec((1, I, H), lambda g, be_r, nb_r: (be_r[g], 0, 0)),
            pl.BlockSpec((1, I, H), lambda g, be_r, nb_r: (be_r[g], 0, 0)),
            pl.BlockSpec((1, H, I), lambda g, be_r, nb_r: (be_r[g], 0, 0)),
        ],
        out_specs=pl.BlockSpec(
            (BT, H), lambda g, be_r, nb_r: (jnp.minimum(g, nb_r[0] - 1), 0)),
    )
    return pl.pallas_call(
        _ffn_body,
        grid_spec=grid_spec,
        out_shape=jax.ShapeDtypeStruct((P, H), jnp.float32),
    )(be, nbv, xs, gate_w, up_w, down_w)


def _shared_body(x_ref, gx_ref, sg_ref, su_ref, sd_ref, o_ref):
    xb = x_ref[...]
    sg = lax.dot_general(xb, sg_ref[...], (((1,), (1,)), ((), ())),
                         preferred_element_type=jnp.float32)
    su = lax.dot_general(xb, su_ref[...], (((1,), (1,)), ((), ())),
                         preferred_element_type=jnp.float32)
    h = sg * lax.logistic(sg) * su
    y = lax.dot_general(h, sd_ref[...], (((1,), (1,)), ((), ())),
                        preferred_element_type=jnp.float32)
    o_ref[...] = y + gx_ref[...]


def _shared_call(flat, gexp, sgw, suw, sdw):
    nblk = T // BT3
    return pl.pallas_call(
        _shared_body,
        grid=(nblk,),
        in_specs=[
            pl.BlockSpec((BT3, H), lambda g: (g, 0)),
            pl.BlockSpec((BT3, H), lambda g: (g, 0)),
            pl.BlockSpec((I_SH, H), lambda g: (0, 0)),
            pl.BlockSpec((I_SH, H), lambda g: (0, 0)),
            pl.BlockSpec((H, I_SH), lambda g: (0, 0)),
        ],
        out_specs=pl.BlockSpec((BT3, H), lambda g: (g, 0)),
        out_shape=jax.ShapeDtypeStruct((T, H), jnp.float32),
    )(flat, gexp, sgw, suw, sdw)


def kernel(x, router_w, gate_w, up_w, down_w,
           shared_gate_w, shared_up_w, shared_down_w):
    Bb, Tt, Hd = x.shape
    flat = x.reshape(T, H)

    dest2, be2, nbv2, aux2 = _router_call(flat, router_w)
    dest = dest2.reshape(T)
    be = be2.reshape(NB)
    nbv = nbv2.reshape(1)
    aux = aux2.reshape(())

    xs = _sc_scatter_kernel()(flat, dest)
    ys = _ffn_call(be, nbv, xs, gate_w, up_w, down_w)
    gexp = _sc_gather_kernel()(ys, dest)
    out = _shared_call(flat, gexp, shared_gate_w, shared_up_w, shared_down_w)
    return out.reshape(Bb, Tt, Hd), aux


# speedup vs baseline: 1.0113x; 1.0113x over previous
import functools

import jax
import jax.numpy as jnp
from jax import lax
from jax.experimental import pallas as pl
from jax.experimental.pallas import tpu as pltpu
from jax.experimental.pallas import tpu_sc as plsc

E = 16
H = 768
I = 512
I_SH = 256
T = 2048
AUX_COEFF = 0.01

BT = 256
BTSHIFT = 8
NB = 24
P = NB * BT
BT3 = 512

_NC = 2
_NS = 16
_NW = _NC * _NS
_TPW = T // _NW


def _router_body(x_ref, rw_ref, dest_ref, be_ref, nbv_ref, aux_ref):
    xf = x_ref[...]
    rw = rw_ref[...]
    logits = lax.dot_general(xf, rw, (((1,), (1,)), ((), ())),
                             preferred_element_type=jnp.float32)
    lmax = jnp.max(logits, axis=1, keepdims=True)
    ex = jnp.exp(logits - lmax)
    probs = ex / jnp.sum(ex, axis=1, keepdims=True)

    ei = lax.broadcasted_iota(jnp.int32, (T, E), 1)
    is_max = logits == lmax
    eid = jnp.min(jnp.where(is_max, ei, E), axis=1, keepdims=True)
    onehot = (ei == eid).astype(jnp.int32)

    counts = jnp.sum(onehot, axis=0, keepdims=True)

    cum = onehot
    k = 1
    while k < T:
        cum = cum + jnp.concatenate(
            [jnp.zeros((k, E), jnp.int32), cum[: T - k, :]], axis=0)
        k *= 2
    rank = jnp.sum(onehot * cum, axis=1, keepdims=True) - 1

    nb_e = lax.shift_right_logical(counts + (BT - 1), BTSHIFT)
    cnb = nb_e
    k = 1
    while k < E:
        cnb = cnb + jnp.concatenate(
            [jnp.zeros((1, k), jnp.int32), cnb[:, : E - k]], axis=1)
        k *= 2
    poffset = lax.shift_left(cnb - nb_e, BTSHIFT)
    nb_total = jnp.max(cnb, axis=1, keepdims=True)

    dest = jnp.sum(onehot * poffset, axis=1, keepdims=True) + rank

    e1 = lax.broadcasted_iota(jnp.int32, (1, E), 1)
    last_e = jnp.max(jnp.where(counts > 0, e1, 0), axis=1, keepdims=True)
    g_iota = lax.broadcasted_iota(jnp.int32, (NB, E), 0)
    cnb_b = jnp.broadcast_to(cnb, (NB, E))
    raw = jnp.sum((g_iota >= cnb_b).astype(jnp.int32), axis=1, keepdims=True)
    be = jnp.minimum(raw, last_e)

    fsum = jnp.sum(counts.astype(jnp.float32) * jnp.sum(probs, axis=0,
                                                        keepdims=True),
                   axis=1, keepdims=True)
    aux = (AUX_COEFF * E / (T * T)) * fsum

    dest_ref[...] = dest
    be_ref[...] = be
    nbv_ref[...] = nb_total
    aux_ref[...] = aux


def _router_call(flat, router_w):
    return pl.pallas_call(
        _router_body,
        out_shape=(
            jax.ShapeDtypeStruct((T, 1), jnp.int32),
            jax.ShapeDtypeStruct((NB, 1), jnp.int32),
            jax.ShapeDtypeStruct((1, 1), jnp.int32),
            jax.ShapeDtypeStruct((1, 1), jnp.float32),
        ),
    )(flat, router_w)


@functools.lru_cache(maxsize=None)
def _sc_scatter_kernel():
    mesh = plsc.VectorSubcoreMesh(core_axis_name="c", subcore_axis_name="s")

    @functools.partial(
        pl.kernel,
        mesh=mesh,
        out_type=jax.ShapeDtypeStruct((P, H), jnp.float32),
        scratch_types=[
            pltpu.VMEM((_TPW,), jnp.int32),
            pltpu.VMEM((_TPW, H), jnp.float32),
            pltpu.SemaphoreType.DMA,
        ],
    )
    def _sc_scatter(x_hbm, dest_hbm, out_hbm, idx_v, rows_v, sem):
        wid = lax.axis_index("s") * _NC + lax.axis_index("c")
        base = wid * _TPW
        pltpu.sync_copy(dest_hbm.at[pl.ds(base, _TPW)], idx_v)
        pltpu.sync_copy(x_hbm.at[pl.ds(base, _TPW)], rows_v)
        pltpu.async_copy(rows_v, out_hbm.at[idx_v], sem).wait()

    return _sc_scatter


@functools.lru_cache(maxsize=None)
def _sc_gather_kernel():
    mesh = plsc.VectorSubcoreMesh(core_axis_name="c", subcore_axis_name="s")

    @functools.partial(
        pl.kernel,
        mesh=mesh,
        out_type=jax.ShapeDtypeStruct((T, H), jnp.float32),
        scratch_types=[
            pltpu.VMEM((_TPW,), jnp.int32),
            pltpu.VMEM((_TPW, H), jnp.float32),
            pltpu.SemaphoreType.DMA,
        ],
    )
    def _sc_gather(ys_hbm, dest_hbm, out_hbm, idx_v, rows_v, sem):
        wid = lax.axis_index("s") * _NC + lax.axis_index("c")
        base = wid * _TPW
        pltpu.sync_copy(dest_hbm.at[pl.ds(base, _TPW)], idx_v)
        pltpu.async_copy(ys_hbm.at[idx_v], rows_v, sem).wait()
        pltpu.sync_copy(rows_v, out_hbm.at[pl.ds(base, _TPW)])

    return _sc_gather


def _ffn_body(be_ref, nbv_ref, xs_ref, gw_ref, uw_ref, dw_ref, ys_ref):
    gb = pl.program_id(0)

    @pl.when(gb < nbv_ref[0])
    def _():
        xb = xs_ref[...]
        gw = gw_ref[0]
        uw = uw_ref[0]
        dw = dw_ref[0]
        g = lax.dot_general(xb, gw, (((1,), (1,)), ((), ())),
                            preferred_element_type=jnp.float32)
        u = lax.dot_general(xb, uw, (((1,), (1,)), ((), ())),
                            preferred_element_type=jnp.float32)
        h = g * lax.logistic(g) * u
        y = lax.dot_general(h, dw, (((1,), (1,)), ((), ())),
                            preferred_element_type=jnp.float32)
        ys_ref[...] = y


def _ffn_call(be, nbv, xs, gate_w, up_w, down_w):
    grid_spec = pltpu.PrefetchScalarGridSpec(
        num_scalar_prefetch=2,
        grid=(NB,),
        in_specs=[
            pl.BlockSpec((BT, H),
                         lambda g, be_r, nb_r: (jnp.minimum(g, nb_r[0] - 1), 0)),
            pl.BlockSpec((1, I, H), lambda g, be_r, nb_r: (be_r[g], 0, 0)),
            pl.BlockSpec((1, I, H), lambda g, be_r, nb_r: (be_r[g], 0, 0)),
            pl.BlockSpec((1, H, I), lambda g, be_r, nb_r: (be_r[g], 0, 0)),
        ],
        out_specs=pl.BlockSpec(
            (BT, H), lambda g, be_r, nb_r: (jnp.minimum(g, nb_r[0] - 1), 0)),
    )
    return pl.pallas_call(
        _ffn_body,
        grid_spec=grid_spec,
        out_shape=jax.ShapeDtypeStruct((P, H), jnp.float32),
    )(be, nbv, xs, gate_w, up_w, down_w)


def _shared_body(x_ref, gx_ref, sg_ref, su_ref, sd_ref, o_ref):
    xb = x_ref[...]
    sg = lax.dot_general(xb, sg_ref[...], (((1,), (1,)), ((), ())),
                         preferred_element_type=jnp.float32)
    su = lax.dot_general(xb, su_ref[...], (((1,), (1,)), ((), ())),
                         preferred_element_type=jnp.float32)
    h = sg * lax.logistic(sg) * su
    y = lax.dot_general(h, sd_ref[...], (((1,), (1,)), ((), ())),
                        preferred_element_type=jnp.float32)
    o_ref[...] = y + gx_ref[...]


def _shared_call(flat, gexp, sgw, suw, sdw):
    nblk = T // BT3
    return pl.pallas_call(
        _shared_body,
        grid=(nblk,),
        in_specs=[
            pl.BlockSpec((BT3, H), lambda g: (g, 0)),
            pl.BlockSpec((BT3, H), lambda g: (g, 0)),
            pl.BlockSpec((I_SH, H), lambda g: (0, 0)),
            pl.BlockSpec((I_SH, H), lambda g: (0, 0)),
            pl.BlockSpec((H, I_SH), lambda g: (0, 0)),
        ],
        out_specs=pl.BlockSpec((BT3, H), lambda g: (g, 0)),
        out_shape=jax.ShapeDtypeStruct((T, H), jnp.float32),
    )(flat, gexp, sgw, suw, sdw)


def kernel(x, router_w, gate_w, up_w, down_w,
           shared_gate_w, shared_up_w, shared_down_w):
    Bb, Tt, Hd = x.shape
    flat = x.reshape(T, H)

    dest2, be2, nbv2, aux2 = _router_call(flat, router_w)
    dest = dest2.reshape(T)
    be = be2.reshape(NB)
    nbv = nbv2.reshape(1)
    aux = aux2.reshape(())

    xs = _sc_scatter_kernel()(flat, dest)
    ys = _ffn_call(be, nbv, xs, gate_w, up_w, down_w)
    gexp = _sc_gather_kernel()(ys, dest)
    out = _shared_call(flat, gexp, shared_gate_w, shared_up_w, shared_down_w)
    return out.reshape(Bb, Tt, Hd), aux
